# baseline (speedup 1.0000x reference)
import functools

import jax
import jax.numpy as jnp
from jax import lax
from jax.experimental import pallas as pl
from jax.experimental.pallas import tpu as pltpu
from jax.experimental.pallas import tpu_sc as plsc

NC, NS = 2, 16
NW = NC * NS
CHUNK = 128
NBUF = 8
DELAY = 4


def _emb_body(idx_hbm, table_hbm, out_hbm, idx_v, rows_v, gsem, wsem, *, g_per_w):
    wid = lax.axis_index("s") * NC + lax.axis_index("c")
    row0 = wid * g_per_w
    pltpu.sync_copy(idx_hbm.at[pl.ds(row0, g_per_w)], idx_v)

    def _wait_gather(s):
        pltpu.make_async_copy(
            table_hbm.at[pl.ds(0, CHUNK)], rows_v.at[s], gsem.at[s]
        ).wait()

    def _wait_write(s):
        pltpu.make_async_copy(rows_v.at[s], out_hbm.at[row0], wsem.at[s]).wait()

    @pl.loop(0, g_per_w, step=NBUF)
    def _(g0):
        for b in range(NBUF):
            g = g0 + b
            s = b

            @pl.when(g >= NBUF)
            def _():
                _wait_write(s)

            pltpu.async_copy(
                table_hbm.at[idx_v.at[g, pl.ds(0, CHUNK // 2)]],
                rows_v.at[s, 0],
                gsem.at[s],
            )
            pltpu.async_copy(
                table_hbm.at[idx_v.at[g, pl.ds(CHUNK // 2, CHUNK // 2)]],
                rows_v.at[s, 1],
                gsem.at[s],
            )

            d = g - DELAY
            sd = (b - DELAY) % NBUF

            @pl.when(d >= 0)
            def _():
                _wait_gather(sd)
                pltpu.async_copy(
                    rows_v.at[sd, 0], out_hbm.at[row0 + d, :, pl.ds(0, 64)],
                    wsem.at[sd],
                )
                pltpu.async_copy(
                    rows_v.at[sd, 1], out_hbm.at[row0 + d, :, pl.ds(64, 64)],
                    wsem.at[sd],
                )

    for e in range(DELAY):
        d = g_per_w - DELAY + e
        sd = d % NBUF
        _wait_gather(sd)
        pltpu.async_copy(
            rows_v.at[sd, 0], out_hbm.at[row0 + d, :, pl.ds(0, 64)], wsem.at[sd]
        )
        pltpu.async_copy(
            rows_v.at[sd, 1], out_hbm.at[row0 + d, :, pl.ds(64, 64)], wsem.at[sd]
        )
    for s in range(NBUF):
        _wait_write(s)


def kernel(inputs, weight):
    bsz, hist = inputs.shape
    vocab, dim = weight.shape
    total = bsz * hist
    assert total % (CHUNK * NW) == 0 and dim == 64
    n_chunks = total // CHUNK
    g_per_w = n_chunks // NW
    assert g_per_w % NBUF == 0

    idx = inputs.reshape(n_chunks, CHUNK).astype(jnp.int32)
    qd = jnp.concatenate([idx[:, 0::2], idx[:, 1::2]], axis=1)

    run = pl.kernel(
        functools.partial(_emb_body, g_per_w=g_per_w),
        out_type=jax.ShapeDtypeStruct((n_chunks, CHUNK // 2, 2 * dim), jnp.float32),
        mesh=plsc.VectorSubcoreMesh(
            core_axis_name="c", subcore_axis_name="s",
            num_cores=NC, num_subcores=NS,
        ),
        scratch_types=[
            pltpu.VMEM((g_per_w, CHUNK), jnp.int32),
            pltpu.VMEM((NBUF, 2, CHUNK // 2, dim), jnp.float32),
            pltpu.SemaphoreType.DMA((NBUF,)),
            pltpu.SemaphoreType.DMA((NBUF,)),
        ],
        compiler_params=pltpu.CompilerParams(use_tc_tiling_on_sc=False),
    )
    out = run(qd, weight)
    return out.reshape(bsz, hist, dim)
